# baseline (speedup 1.0000x reference)
import functools

import jax
import jax.numpy as jnp
from jax import lax
from jax.experimental import pallas as pl
from jax.experimental.pallas import tpu as pltpu
from jax.experimental.pallas import tpu_sc as plsc

_N = 10000
_E = 320000
_NPAD = 10240
_G = _E // 128
_NC = 2
_NS = 16
_NW = _NC * _NS
_GW = _G // _NW
_GMAX = _GW + 1
_NFULL = (_GW // 4) * 4
_ROWS_PER_TILE = _NPAD // _NS
_NBUF = 8

_mesh = plsc.VectorSubcoreMesh(core_axis_name="c", subcore_axis_name="s")


def _fill(ref, n16, value):
    def body(i, carry):
        ref[pl.ds(i * 16, 16)] = jnp.full((16,), value, jnp.float32)
        return carry
    lax.fori_loop(0, n16, body, 0)


def _worker_groups(wid):
    g0 = wid * _GW + jnp.minimum(wid, _G - _GW * _NW)
    ng = _GW + jnp.where(wid < _G - _GW * _NW, 1, 0)
    return g0, ng


def _load_groups(edge_hbm, row, g0, ng, idx, sem):
    def issue(j, carry):
        pltpu.async_copy(
            edge_hbm.at[row, pl.ds((g0 + j) * 128, 128)], idx.at[j], sem)
        return carry
    lax.fori_loop(0, ng, issue, 0)

    def drain(j, carry):
        pltpu.make_async_copy(
            edge_hbm.at[row, pl.ds(g0 * 128, 128)], idx.at[0], sem).wait()
        return carry
    lax.fori_loop(0, ng, drain, 0)


@functools.partial(
    pl.kernel,
    out_type=jax.ShapeDtypeStruct((_NC, _NPAD), jnp.float32),
    mesh=_mesh,
    compiler_params=pltpu.CompilerParams(use_tc_tiling_on_sc=False),
    scratch_types=[
        pltpu.VMEM_SHARED((_NPAD,), jnp.float32),
        pltpu.VMEM((_GMAX, 128), jnp.int32),
        pltpu.VMEM((_ROWS_PER_TILE,), jnp.float32),
        pltpu.VMEM((128,), jnp.float32),
        pltpu.SemaphoreType.DMA,
    ],
)
def _deg_kernel(edge_hbm, out_hbm, deg_sh, didx, zbuf, ones, sem):
    c = lax.axis_index("c")
    s = lax.axis_index("s")
    wid = c * _NS + s
    g0, ng = _worker_groups(wid)
    _load_groups(edge_hbm, 1, g0, ng, didx, sem)
    _fill(zbuf, _ROWS_PER_TILE // 16, 0.0)
    _fill(ones, 128 // 16, 1.0)
    pltpu.sync_copy(zbuf, deg_sh.at[pl.ds(s * _ROWS_PER_TILE, _ROWS_PER_TILE)])
    plsc.subcore_barrier()

    def body(g, carry):
        descs = []
        for k in range(4):
            descs.append(pltpu.async_copy(
                ones, deg_sh.at[didx.at[g * 4 + k]], sem, add=True))
        for d in descs:
            d.wait()
        return carry
    lax.fori_loop(0, _NFULL // 4, body, 0)

    def tail(g, carry):
        pltpu.sync_copy(ones, deg_sh.at[didx.at[g]], add=True)
        return carry
    lax.fori_loop(_NFULL, ng, tail, 0)

    plsc.subcore_barrier()
    pltpu.sync_copy(
        deg_sh.at[pl.ds(s * _ROWS_PER_TILE, _ROWS_PER_TILE)],
        out_hbm.at[c, pl.ds(s * _ROWS_PER_TILE, _ROWS_PER_TILE)],
    )


@functools.partial(
    pl.kernel,
    out_type=jax.ShapeDtypeStruct((_NC, _NPAD, 64), jnp.float32),
    mesh=_mesh,
    compiler_params=pltpu.CompilerParams(use_tc_tiling_on_sc=False),
    scratch_types=[
        pltpu.VMEM_SHARED((_NPAD, 64), jnp.float32),
        pltpu.VMEM((_GMAX, 128), jnp.int32),
        pltpu.VMEM((_GMAX, 128), jnp.int32),
        pltpu.VMEM((_NBUF, 128, 64), jnp.float32),
        pltpu.SemaphoreType.DMA,
        pltpu.SemaphoreType.DMA,
        pltpu.SemaphoreType.DMA,
    ],
)
def _agg_kernel(h2_hbm, edge_hbm, out_hbm, agg_sh, sidx, didx, rows,
                sem_i, sem_g, sem_s):
    c = lax.axis_index("c")
    s = lax.axis_index("s")
    wid = c * _NS + s
    g0, ng = _worker_groups(wid)
    _load_groups(edge_hbm, 0, g0, ng, sidx, sem_i)
    _load_groups(edge_hbm, 1, g0, ng, didx, sem_i)

    def zrow(i, carry):
        r = i // 4
        k = i % 4
        rows[0, r, pl.ds(k * 16, 16)] = jnp.zeros((16,), jnp.float32)
        return carry
    lax.fori_loop(0, 512, zrow, 0)
    for m in range(_ROWS_PER_TILE // 128):
        pltpu.sync_copy(
            rows.at[0], agg_sh.at[pl.ds(s * _ROWS_PER_TILE + m * 128, 128)])
    plsc.subcore_barrier()

    kb = _NBUF // 2
    for k in range(2 * kb):
        pltpu.async_copy(h2_hbm.at[sidx.at[k]], rows.at[k], sem_g)

    def body(hg, carry):
        bank = lax.rem(hg, 2)
        base = hg * kb
        slot0 = bank * kb
        for k in range(kb):
            pltpu.make_async_copy(
                h2_hbm.at[sidx.at[base + k]], rows.at[slot0 + k], sem_g
            ).wait()
        sd = []
        for k in range(kb):
            sd.append(pltpu.async_copy(
                rows.at[slot0 + k], agg_sh.at[didx.at[base + k]], sem_s,
                add=True))
        for d in sd:
            d.wait()

        @pl.when(hg < _NFULL // kb - 2)
        def _prefetch():
            nb = base + 2 * kb
            for k in range(kb):
                pltpu.async_copy(
                    h2_hbm.at[sidx.at[nb + k]], rows.at[slot0 + k], sem_g)
        return carry
    lax.fori_loop(0, _NFULL // kb, body, 0)

    def tail(g, carry):
        pltpu.async_copy(h2_hbm.at[sidx.at[g]], rows.at[0], sem_g).wait()
        pltpu.sync_copy(rows.at[0], agg_sh.at[didx.at[g]], add=True)
        return carry
    lax.fori_loop(_NFULL, ng, tail, 0)

    plsc.subcore_barrier()
    pltpu.sync_copy(
        agg_sh.at[pl.ds(s * _ROWS_PER_TILE, _ROWS_PER_TILE)],
        out_hbm.at[c, pl.ds(s * _ROWS_PER_TILE, _ROWS_PER_TILE)],
    )


_BN = 2048


def _h_body(x_ref, w_ref, h_ref):
    h_ref[...] = jnp.dot(x_ref[...], w_ref[...],
                         preferred_element_type=jnp.float32)


def _degcol(dp_ref, i):
    row = (dp_ref[0:1, pl.ds(i * _BN, _BN)]
           + dp_ref[1:2, pl.ds(i * _BN, _BN)] + 1.0)
    return jnp.transpose(row, (1, 0))


def _scale_body(h_ref, dp_ref, h2_ref):
    i = pl.program_id(0)
    h2_ref[...] = h_ref[...] * lax.rsqrt(_degcol(dp_ref, i))


def _tail_body(aggp_hbm, h2_ref, dpt_ref, dist_ref, bg_ref, we1_ref, be1_ref,
               we2_ref, be2_ref, woh_ref, woe_ref, bo_ref, out_ref, abuf, sem):
    i = pl.program_id(0)
    for cc in range(_NC):
        pltpu.async_copy(
            aggp_hbm.at[cc, pl.ds(i * _BN, _BN)], abuf.at[cc], sem).wait()
    agg = abuf[0] + abuf[1]
    d = lax.rsqrt(_degcol(dpt_ref, pl.program_id(0)))
    gcn = d * (agg + h2_ref[...]) + bg_ref[...]
    enc = jnp.dot(dist_ref[...], we1_ref[...],
                  preferred_element_type=jnp.float32) + be1_ref[...]
    enc = jnp.maximum(enc, 0.0)
    enc = jnp.dot(enc, we2_ref[...],
                  preferred_element_type=jnp.float32) + be2_ref[...]
    o = (jnp.dot(gcn, woh_ref[...], preferred_element_type=jnp.float32)
         + jnp.dot(enc, woe_ref[...], preferred_element_type=jnp.float32)
         + bo_ref[...])
    m = jnp.max(o, axis=1, keepdims=True)
    sh = o - m
    lse = jnp.log(jnp.sum(jnp.exp(sh), axis=1, keepdims=True))
    out_ref[...] = sh - lse


def kernel(x, edge_index, batch, distances, W_gcn, b_gcn, W_enc1, b_enc1,
           W_enc2, b_enc2, W_out, b_out):
    del batch
    deg_p = _deg_kernel(edge_index)
    h = pl.pallas_call(
        _h_body,
        grid=(_NPAD // _BN,),
        in_specs=[
            pl.BlockSpec((_BN, 128), lambda i: (i, 0)),
            pl.BlockSpec((128, 64), lambda i: (0, 0)),
        ],
        out_specs=pl.BlockSpec((_BN, 64), lambda i: (i, 0)),
        out_shape=jax.ShapeDtypeStruct((_NPAD, 64), jnp.float32),
    )(x, W_gcn)

    h2 = pl.pallas_call(
        _scale_body,
        grid=(_NPAD // _BN,),
        in_specs=[
            pl.BlockSpec((_BN, 64), lambda i: (i, 0)),
            pl.BlockSpec((_NC, _NPAD), lambda i: (0, 0)),
        ],
        out_specs=pl.BlockSpec((_BN, 64), lambda i: (i, 0)),
        out_shape=jax.ShapeDtypeStruct((_NPAD, 64), jnp.float32),
    )(h, deg_p)

    aggp = _agg_kernel(h2, edge_index)

    out = pl.pallas_call(
        _tail_body,
        grid=(_NPAD // _BN,),
        in_specs=[
            pl.BlockSpec(memory_space=pl.ANY),
            pl.BlockSpec((_BN, 64), lambda i: (i, 0)),
            pl.BlockSpec((_NC, _NPAD), lambda i: (0, 0)),
            pl.BlockSpec((_BN, 2), lambda i: (i, 0)),
            pl.BlockSpec((1, 64), lambda i: (0, 0)),
            pl.BlockSpec((2, 32), lambda i: (0, 0)),
            pl.BlockSpec((1, 32), lambda i: (0, 0)),
            pl.BlockSpec((32, 32), lambda i: (0, 0)),
            pl.BlockSpec((1, 32), lambda i: (0, 0)),
            pl.BlockSpec((64, 16), lambda i: (0, 0)),
            pl.BlockSpec((32, 16), lambda i: (0, 0)),
            pl.BlockSpec((1, 16), lambda i: (0, 0)),
        ],
        out_specs=pl.BlockSpec((_BN, 16), lambda i: (i, 0)),
        out_shape=jax.ShapeDtypeStruct((_N, 16), jnp.float32),
        scratch_shapes=[
            pltpu.VMEM((_NC, _BN, 64), jnp.float32),
            pltpu.SemaphoreType.DMA,
        ],
    )(aggp, h2, deg_p, distances, b_gcn.reshape(1, 64), W_enc1,
      b_enc1.reshape(1, 32), W_enc2, b_enc2.reshape(1, 32), W_out[:64],
      W_out[64:], b_out.reshape(1, 16))

    return out


# speedup vs baseline: 1.0991x; 1.0991x over previous
import functools

import jax
import jax.numpy as jnp
from jax import lax
from jax.experimental import pallas as pl
from jax.experimental.pallas import tpu as pltpu
from jax.experimental.pallas import tpu_sc as plsc

_N = 10000
_E = 320000
_NPAD = 10240
_G = _E // 128
_NC = 2
_NS = 16
_NW = _NC * _NS
_GW = _G // _NW
_GMAX = _GW + 1
_NFULL = (_GW // 4) * 4
_ROWS_PER_TILE = _NPAD // _NS
_NBUF = 8

_mesh = plsc.VectorSubcoreMesh(core_axis_name="c", subcore_axis_name="s")


def _fill(ref, n16, value):
    def body(i, carry):
        ref[pl.ds(i * 16, 16)] = jnp.full((16,), value, jnp.float32)
        return carry
    lax.fori_loop(0, n16, body, 0)


def _worker_groups(wid):
    g0 = wid * _GW + jnp.minimum(wid, _G - _GW * _NW)
    ng = _GW + jnp.where(wid < _G - _GW * _NW, 1, 0)
    return g0, ng


def _load_groups(edge_hbm, row, g0, ng, idx, sem):
    def issue(j, carry):
        pltpu.async_copy(
            edge_hbm.at[row, pl.ds((g0 + j) * 128, 128)], idx.at[j], sem)
        return carry
    lax.fori_loop(0, ng, issue, 0)

    def drain(j, carry):
        pltpu.make_async_copy(
            edge_hbm.at[row, pl.ds(g0 * 128, 128)], idx.at[0], sem).wait()
        return carry
    lax.fori_loop(0, ng, drain, 0)


@functools.partial(
    pl.kernel,
    out_type=jax.ShapeDtypeStruct((_NC, _NPAD), jnp.float32),
    mesh=_mesh,
    compiler_params=pltpu.CompilerParams(use_tc_tiling_on_sc=False),
    scratch_types=[
        pltpu.VMEM_SHARED((_NPAD,), jnp.float32),
        pltpu.VMEM((_GMAX, 128), jnp.int32),
        pltpu.VMEM((_ROWS_PER_TILE,), jnp.float32),
        pltpu.VMEM((128,), jnp.float32),
        pltpu.SemaphoreType.DMA,
    ],
)
def _deg_kernel(edge_hbm, out_hbm, deg_sh, didx, zbuf, ones, sem):
    c = lax.axis_index("c")
    s = lax.axis_index("s")
    wid = c * _NS + s
    g0, ng = _worker_groups(wid)
    _load_groups(edge_hbm, 1, g0, ng, didx, sem)
    _fill(zbuf, _ROWS_PER_TILE // 16, 0.0)
    _fill(ones, 128 // 16, 1.0)
    pltpu.sync_copy(zbuf, deg_sh.at[pl.ds(s * _ROWS_PER_TILE, _ROWS_PER_TILE)])
    plsc.subcore_barrier()

    def body(g, carry):
        descs = []
        for k in range(4):
            descs.append(pltpu.async_copy(
                ones, deg_sh.at[didx.at[g * 4 + k]], sem, add=True))
        for d in descs:
            d.wait()
        return carry
    lax.fori_loop(0, _NFULL // 4, body, 0)

    def tail(g, carry):
        pltpu.sync_copy(ones, deg_sh.at[didx.at[g]], add=True)
        return carry
    lax.fori_loop(_NFULL, ng, tail, 0)

    plsc.subcore_barrier()
    pltpu.sync_copy(
        deg_sh.at[pl.ds(s * _ROWS_PER_TILE, _ROWS_PER_TILE)],
        out_hbm.at[c, pl.ds(s * _ROWS_PER_TILE, _ROWS_PER_TILE)],
    )


@functools.partial(
    pl.kernel,
    out_type=jax.ShapeDtypeStruct((_NC, _NPAD, 64), jnp.float32),
    mesh=_mesh,
    compiler_params=pltpu.CompilerParams(use_tc_tiling_on_sc=False),
    scratch_types=[
        pltpu.VMEM_SHARED((_NPAD, 64), jnp.float32),
        pltpu.VMEM((_GMAX, 128), jnp.int32),
        pltpu.VMEM((_GMAX, 128), jnp.int32),
        pltpu.VMEM((_NBUF, 128, 64), jnp.float32),
        pltpu.SemaphoreType.DMA,
        pltpu.SemaphoreType.DMA,
        pltpu.SemaphoreType.DMA,
    ],
)
def _agg_kernel(h2_hbm, edge_hbm, out_hbm, agg_sh, sidx, didx, rows,
                sem_i, sem_g, sem_s):
    c = lax.axis_index("c")
    s = lax.axis_index("s")
    wid = c * _NS + s
    g0, ng = _worker_groups(wid)
    _load_groups(edge_hbm, 0, g0, ng, sidx, sem_i)
    _load_groups(edge_hbm, 1, g0, ng, didx, sem_i)

    def zrow(i, carry):
        r = i // 4
        k = i % 4
        rows[0, r, pl.ds(k * 16, 16)] = jnp.zeros((16,), jnp.float32)
        return carry
    lax.fori_loop(0, 512, zrow, 0)
    for m in range(_ROWS_PER_TILE // 128):
        pltpu.sync_copy(
            rows.at[0], agg_sh.at[pl.ds(s * _ROWS_PER_TILE + m * 128, 128)])
    plsc.subcore_barrier()

    kb = 4
    nbank = _NBUF // kb
    for k in range(nbank * kb):
        pltpu.async_copy(h2_hbm.at[sidx.at[k]], rows.at[k], sem_g)

    def body(hg, carry):
        bank = lax.rem(hg, nbank)
        base = hg * kb
        slot0 = bank * kb
        for k in range(kb):
            pltpu.make_async_copy(
                h2_hbm.at[sidx.at[base + k]], rows.at[slot0 + k], sem_g
            ).wait()
        sd = []
        for k in range(kb):
            sd.append(pltpu.async_copy(
                rows.at[slot0 + k], agg_sh.at[didx.at[base + k]], sem_s,
                add=True))
        for d in sd:
            d.wait()

        @pl.when(hg < _NFULL // kb - nbank)
        def _prefetch():
            nb = base + nbank * kb
            for k in range(kb):
                pltpu.async_copy(
                    h2_hbm.at[sidx.at[nb + k]], rows.at[slot0 + k], sem_g)
        return carry
    lax.fori_loop(0, _NFULL // kb, body, 0)

    def tail(g, carry):
        pltpu.async_copy(h2_hbm.at[sidx.at[g]], rows.at[0], sem_g).wait()
        pltpu.sync_copy(rows.at[0], agg_sh.at[didx.at[g]], add=True)
        return carry
    lax.fori_loop(_NFULL, ng, tail, 0)

    plsc.subcore_barrier()
    pltpu.sync_copy(
        agg_sh.at[pl.ds(s * _ROWS_PER_TILE, _ROWS_PER_TILE)],
        out_hbm.at[c, pl.ds(s * _ROWS_PER_TILE, _ROWS_PER_TILE)],
    )


_BN = 2048


def _h_body(x_ref, w_ref, h_ref):
    h_ref[...] = jnp.dot(x_ref[...], w_ref[...],
                         preferred_element_type=jnp.float32)


def _degcol(dp_ref, i):
    row = (dp_ref[0:1, pl.ds(i * _BN, _BN)]
           + dp_ref[1:2, pl.ds(i * _BN, _BN)] + 1.0)
    return jnp.transpose(row, (1, 0))


def _scale_body(h_ref, dp_ref, h2_ref):
    i = pl.program_id(0)
    h2_ref[...] = h_ref[...] * lax.rsqrt(_degcol(dp_ref, i))


def _tail_body(aggp_ref, h2_ref, dpt_ref, dist_ref, bg_ref, we1_ref, be1_ref,
               we2_ref, be2_ref, woh_ref, woe_ref, bo_ref, out_ref):
    agg = aggp_ref[0] + aggp_ref[1]
    d = lax.rsqrt(_degcol(dpt_ref, pl.program_id(0)))
    gcn = d * (agg + h2_ref[...]) + bg_ref[...]
    enc = jnp.dot(dist_ref[...], we1_ref[...],
                  preferred_element_type=jnp.float32) + be1_ref[...]
    enc = jnp.maximum(enc, 0.0)
    enc = jnp.dot(enc, we2_ref[...],
                  preferred_element_type=jnp.float32) + be2_ref[...]
    o = (jnp.dot(gcn, woh_ref[...], preferred_element_type=jnp.float32)
         + jnp.dot(enc, woe_ref[...], preferred_element_type=jnp.float32)
         + bo_ref[...])
    m = jnp.max(o, axis=1, keepdims=True)
    sh = o - m
    lse = jnp.log(jnp.sum(jnp.exp(sh), axis=1, keepdims=True))
    out_ref[...] = sh - lse


def kernel(x, edge_index, batch, distances, W_gcn, b_gcn, W_enc1, b_enc1,
           W_enc2, b_enc2, W_out, b_out):
    del batch
    deg_p = _deg_kernel(edge_index)
    h = pl.pallas_call(
        _h_body,
        grid=(_NPAD // _BN,),
        in_specs=[
            pl.BlockSpec((_BN, 128), lambda i: (i, 0)),
            pl.BlockSpec((128, 64), lambda i: (0, 0)),
        ],
        out_specs=pl.BlockSpec((_BN, 64), lambda i: (i, 0)),
        out_shape=jax.ShapeDtypeStruct((_NPAD, 64), jnp.float32),
    )(x, W_gcn)

    h2 = pl.pallas_call(
        _scale_body,
        grid=(_NPAD // _BN,),
        in_specs=[
            pl.BlockSpec((_BN, 64), lambda i: (i, 0)),
            pl.BlockSpec((_NC, _NPAD), lambda i: (0, 0)),
        ],
        out_specs=pl.BlockSpec((_BN, 64), lambda i: (i, 0)),
        out_shape=jax.ShapeDtypeStruct((_NPAD, 64), jnp.float32),
    )(h, deg_p)

    aggp = _agg_kernel(h2, edge_index)

    out = pl.pallas_call(
        _tail_body,
        grid=(_NPAD // _BN,),
        in_specs=[
            pl.BlockSpec((_NC, _BN, 64), lambda i: (0, i, 0)),
            pl.BlockSpec((_BN, 64), lambda i: (i, 0)),
            pl.BlockSpec((_NC, _NPAD), lambda i: (0, 0)),
            pl.BlockSpec((_BN, 2), lambda i: (i, 0)),
            pl.BlockSpec((1, 64), lambda i: (0, 0)),
            pl.BlockSpec((2, 32), lambda i: (0, 0)),
            pl.BlockSpec((1, 32), lambda i: (0, 0)),
            pl.BlockSpec((32, 32), lambda i: (0, 0)),
            pl.BlockSpec((1, 32), lambda i: (0, 0)),
            pl.BlockSpec((64, 16), lambda i: (0, 0)),
            pl.BlockSpec((32, 16), lambda i: (0, 0)),
            pl.BlockSpec((1, 16), lambda i: (0, 0)),
        ],
        out_specs=pl.BlockSpec((_BN, 16), lambda i: (i, 0)),
        out_shape=jax.ShapeDtypeStruct((_N, 16), jnp.float32),
    )(aggp, h2, deg_p, distances, b_gcn.reshape(1, 64), W_enc1,
      b_enc1.reshape(1, 32), W_enc2, b_enc2.reshape(1, 32), W_out[:64],
      W_out[64:], b_out.reshape(1, 16))

    return out


# speedup vs baseline: 1.1368x; 1.0343x over previous
import functools

import jax
import jax.numpy as jnp
from jax import lax
from jax.experimental import pallas as pl
from jax.experimental.pallas import tpu as pltpu
from jax.experimental.pallas import tpu_sc as plsc

_N = 10000
_E = 320000
_NPAD = 10240
_G = _E // 128
_NC = 2
_NS = 16
_NW = _NC * _NS
_GW = _G // _NW
_GMAX = _GW + 1
_NFULL = (_GW // 4) * 4
_ROWS_PER_TILE = _NPAD // _NS
_NBUF = 8

_mesh = plsc.VectorSubcoreMesh(core_axis_name="c", subcore_axis_name="s")


def _fill(ref, n16, value):
    def body(i, carry):
        ref[pl.ds(i * 16, 16)] = jnp.full((16,), value, jnp.float32)
        return carry
    lax.fori_loop(0, n16, body, 0)


def _worker_groups(wid):
    g0 = wid * _GW + jnp.minimum(wid, _G - _GW * _NW)
    ng = _GW + jnp.where(wid < _G - _GW * _NW, 1, 0)
    return g0, ng


def _load_groups(edge_hbm, row, g0, ng, idx, sem):
    def issue(j, carry):
        pltpu.async_copy(
            edge_hbm.at[row, pl.ds((g0 + j) * 128, 128)], idx.at[j], sem)
        return carry
    lax.fori_loop(0, ng, issue, 0)

    def drain(j, carry):
        pltpu.make_async_copy(
            edge_hbm.at[row, pl.ds(g0 * 128, 128)], idx.at[0], sem).wait()
        return carry
    lax.fori_loop(0, ng, drain, 0)


@functools.partial(
    pl.kernel,
    out_type=jax.ShapeDtypeStruct((_NC, _NPAD), jnp.float32),
    mesh=_mesh,
    compiler_params=pltpu.CompilerParams(use_tc_tiling_on_sc=False),
    scratch_types=[
        pltpu.VMEM_SHARED((_NPAD,), jnp.float32),
        pltpu.VMEM((_GMAX, 128), jnp.int32),
        pltpu.VMEM((_ROWS_PER_TILE,), jnp.float32),
        pltpu.VMEM((128,), jnp.float32),
        pltpu.SemaphoreType.DMA,
    ],
)
def _deg_kernel(edge_hbm, out_hbm, deg_sh, didx, zbuf, ones, sem):
    c = lax.axis_index("c")
    s = lax.axis_index("s")
    wid = c * _NS + s
    g0, ng = _worker_groups(wid)
    _load_groups(edge_hbm, 1, g0, ng, didx, sem)
    _fill(zbuf, _ROWS_PER_TILE // 16, 0.0)
    _fill(ones, 128 // 16, 1.0)
    pltpu.sync_copy(zbuf, deg_sh.at[pl.ds(s * _ROWS_PER_TILE, _ROWS_PER_TILE)])
    plsc.subcore_barrier()

    def body(g, carry):
        descs = []
        for k in range(4):
            descs.append(pltpu.async_copy(
                ones, deg_sh.at[didx.at[g * 4 + k]], sem, add=True))
        for d in descs:
            d.wait()
        return carry
    lax.fori_loop(0, _NFULL // 4, body, 0)

    def tail(g, carry):
        pltpu.sync_copy(ones, deg_sh.at[didx.at[g]], add=True)
        return carry
    lax.fori_loop(_NFULL, ng, tail, 0)

    plsc.subcore_barrier()
    pltpu.sync_copy(
        deg_sh.at[pl.ds(s * _ROWS_PER_TILE, _ROWS_PER_TILE)],
        out_hbm.at[c, pl.ds(s * _ROWS_PER_TILE, _ROWS_PER_TILE)],
    )


@functools.partial(
    pl.kernel,
    out_type=jax.ShapeDtypeStruct((_NC, _NPAD, 64), jnp.float32),
    mesh=_mesh,
    compiler_params=pltpu.CompilerParams(use_tc_tiling_on_sc=False),
    scratch_types=[
        pltpu.VMEM_SHARED((_NPAD, 64), jnp.float32),
        pltpu.VMEM((_GMAX, 128), jnp.int32),
        pltpu.VMEM((_GMAX, 128), jnp.int32),
        pltpu.VMEM((_NBUF, 128, 64), jnp.float32),
        pltpu.SemaphoreType.DMA,
        pltpu.SemaphoreType.DMA,
        pltpu.SemaphoreType.DMA,
    ],
)
def _agg_kernel(h2_hbm, edge_hbm, out_hbm, agg_sh, sidx, didx, rows,
                sem_i, sem_g, sem_s):
    c = lax.axis_index("c")
    s = lax.axis_index("s")
    wid = c * _NS + s
    g0, ng = _worker_groups(wid)
    _load_groups(edge_hbm, 0, g0, ng, sidx, sem_i)
    _load_groups(edge_hbm, 1, g0, ng, didx, sem_i)

    def zrow(i, carry):
        r = i // 4
        k = i % 4
        rows[0, r, pl.ds(k * 16, 16)] = jnp.zeros((16,), jnp.float32)
        return carry
    lax.fori_loop(0, 512, zrow, 0)
    for m in range(_ROWS_PER_TILE // 128):
        pltpu.sync_copy(
            rows.at[0], agg_sh.at[pl.ds(s * _ROWS_PER_TILE + m * 128, 128)])
    plsc.subcore_barrier()

    kb = 2
    nbank = _NBUF // kb
    for k in range(nbank * kb):
        pltpu.async_copy(h2_hbm.at[sidx.at[k]], rows.at[k], sem_g)

    def body(hg, carry):
        bank = lax.rem(hg, nbank)
        base = hg * kb
        slot0 = bank * kb
        for k in range(kb):
            pltpu.make_async_copy(
                h2_hbm.at[sidx.at[base + k]], rows.at[slot0 + k], sem_g
            ).wait()
        sd = []
        for k in range(kb):
            sd.append(pltpu.async_copy(
                rows.at[slot0 + k], agg_sh.at[didx.at[base + k]], sem_s,
                add=True))
        for d in sd:
            d.wait()

        @pl.when(hg < _NFULL // kb - nbank)
        def _prefetch():
            nb = base + nbank * kb
            for k in range(kb):
                pltpu.async_copy(
                    h2_hbm.at[sidx.at[nb + k]], rows.at[slot0 + k], sem_g)
        return carry
    lax.fori_loop(0, _NFULL // kb, body, 0)

    def tail(g, carry):
        pltpu.async_copy(h2_hbm.at[sidx.at[g]], rows.at[0], sem_g).wait()
        pltpu.sync_copy(rows.at[0], agg_sh.at[didx.at[g]], add=True)
        return carry
    lax.fori_loop(_NFULL, ng, tail, 0)

    plsc.subcore_barrier()
    pltpu.sync_copy(
        agg_sh.at[pl.ds(s * _ROWS_PER_TILE, _ROWS_PER_TILE)],
        out_hbm.at[c, pl.ds(s * _ROWS_PER_TILE, _ROWS_PER_TILE)],
    )


_BN = 2048


def _h_body(x_ref, w_ref, h_ref):
    h_ref[...] = jnp.dot(x_ref[...], w_ref[...],
                         preferred_element_type=jnp.float32)


def _degcol(dp_ref, i):
    row = (dp_ref[0:1, pl.ds(i * _BN, _BN)]
           + dp_ref[1:2, pl.ds(i * _BN, _BN)] + 1.0)
    return jnp.transpose(row, (1, 0))


def _scale_body(h_ref, dp_ref, h2_ref):
    i = pl.program_id(0)
    h2_ref[...] = h_ref[...] * lax.rsqrt(_degcol(dp_ref, i))


def _tail_body(aggp_ref, h2_ref, dpt_ref, dist_ref, bg_ref, we1_ref, be1_ref,
               we2_ref, be2_ref, woh_ref, woe_ref, bo_ref, out_ref):
    agg = aggp_ref[0] + aggp_ref[1]
    d = lax.rsqrt(_degcol(dpt_ref, pl.program_id(0)))
    gcn = d * (agg + h2_ref[...]) + bg_ref[...]
    enc = jnp.dot(dist_ref[...], we1_ref[...],
                  preferred_element_type=jnp.float32) + be1_ref[...]
    enc = jnp.maximum(enc, 0.0)
    enc = jnp.dot(enc, we2_ref[...],
                  preferred_element_type=jnp.float32) + be2_ref[...]
    o = (jnp.dot(gcn, woh_ref[...], preferred_element_type=jnp.float32)
         + jnp.dot(enc, woe_ref[...], preferred_element_type=jnp.float32)
         + bo_ref[...])
    m = jnp.max(o, axis=1, keepdims=True)
    sh = o - m
    lse = jnp.log(jnp.sum(jnp.exp(sh), axis=1, keepdims=True))
    out_ref[...] = sh - lse


def kernel(x, edge_index, batch, distances, W_gcn, b_gcn, W_enc1, b_enc1,
           W_enc2, b_enc2, W_out, b_out):
    del batch
    deg_p = _deg_kernel(edge_index)
    h = pl.pallas_call(
        _h_body,
        grid=(_NPAD // _BN,),
        in_specs=[
            pl.BlockSpec((_BN, 128), lambda i: (i, 0)),
            pl.BlockSpec((128, 64), lambda i: (0, 0)),
        ],
        out_specs=pl.BlockSpec((_BN, 64), lambda i: (i, 0)),
        out_shape=jax.ShapeDtypeStruct((_NPAD, 64), jnp.float32),
    )(x, W_gcn)

    h2 = pl.pallas_call(
        _scale_body,
        grid=(_NPAD // _BN,),
        in_specs=[
            pl.BlockSpec((_BN, 64), lambda i: (i, 0)),
            pl.BlockSpec((_NC, _NPAD), lambda i: (0, 0)),
        ],
        out_specs=pl.BlockSpec((_BN, 64), lambda i: (i, 0)),
        out_shape=jax.ShapeDtypeStruct((_NPAD, 64), jnp.float32),
    )(h, deg_p)

    aggp = _agg_kernel(h2, edge_index)

    out = pl.pallas_call(
        _tail_body,
        grid=(_NPAD // _BN,),
        in_specs=[
            pl.BlockSpec((_NC, _BN, 64), lambda i: (0, i, 0)),
            pl.BlockSpec((_BN, 64), lambda i: (i, 0)),
            pl.BlockSpec((_NC, _NPAD), lambda i: (0, 0)),
            pl.BlockSpec((_BN, 2), lambda i: (i, 0)),
            pl.BlockSpec((1, 64), lambda i: (0, 0)),
            pl.BlockSpec((2, 32), lambda i: (0, 0)),
            pl.BlockSpec((1, 32), lambda i: (0, 0)),
            pl.BlockSpec((32, 32), lambda i: (0, 0)),
            pl.BlockSpec((1, 32), lambda i: (0, 0)),
            pl.BlockSpec((64, 16), lambda i: (0, 0)),
            pl.BlockSpec((32, 16), lambda i: (0, 0)),
            pl.BlockSpec((1, 16), lambda i: (0, 0)),
        ],
        out_specs=pl.BlockSpec((_BN, 16), lambda i: (i, 0)),
        out_shape=jax.ShapeDtypeStruct((_N, 16), jnp.float32),
    )(aggp, h2, deg_p, distances, b_gcn.reshape(1, 64), W_enc1,
      b_enc1.reshape(1, 32), W_enc2, b_enc2.reshape(1, 32), W_out[:64],
      W_out[64:], b_out.reshape(1, 16))

    return out
